# Initial kernel scaffold; baseline (speedup 1.0000x reference)
import jax
import jax.numpy as jnp
from jax.experimental import pallas as pl


def kernel(x, v, edge_index, W1, b1, W2, b2, U1, c1, U2, c2):
    raise NotImplementedError("write your pallas kernel here")


# speedup vs baseline: 2.3272x; 2.3272x over previous
import jax
import jax.numpy as jnp
from jax import lax
from jax.experimental import pallas as pl
from jax.experimental.pallas import tpu as pltpu
from jax.experimental.pallas import tpu_sc as plsc

N = 10000
E = 160000
HD = 128
NF = 512

NC = 2
NS = 16
NW = NC * NS


_RB = 2000


def _pre_body(x_ref, v_ref, w1a_ref, w1b_ref, b1_ref, a_ref, b_ref, p_ref):
    nf = jnp.concatenate([x_ref[...], v_ref[...]], axis=1)
    p0 = jnp.mean(v_ref[:, 0:HD], axis=1, keepdims=True)
    p1 = jnp.mean(v_ref[:, HD:2 * HD], axis=1, keepdims=True)
    p2 = jnp.mean(v_ref[:, 2 * HD:3 * HD], axis=1, keepdims=True)
    pad = jnp.zeros((nf.shape[0], 13), jnp.float32)
    p_ref[...] = jnp.concatenate([p0, p1, p2, pad], axis=1)
    a_ref[...] = jnp.dot(nf, w1a_ref[...],
                         preferred_element_type=jnp.float32) + b1_ref[...]
    b_ref[...] = jnp.dot(nf, w1b_ref[...],
                         preferred_element_type=jnp.float32)


def _pre(x, v2, w1a, w1b, b1r):
    grid = (N // _RB,)
    return pl.pallas_call(
        _pre_body,
        grid=grid,
        in_specs=[
            pl.BlockSpec((_RB, HD), lambda i: (i, 0)),
            pl.BlockSpec((_RB, 3 * HD), lambda i: (i, 0)),
            pl.BlockSpec((NF, NF), lambda i: (0, 0)),
            pl.BlockSpec((NF, NF), lambda i: (0, 0)),
            pl.BlockSpec((1, NF), lambda i: (0, 0)),
        ],
        out_specs=[
            pl.BlockSpec((_RB, NF), lambda i: (i, 0)),
            pl.BlockSpec((_RB, NF), lambda i: (i, 0)),
            pl.BlockSpec((_RB, 16), lambda i: (i, 0)),
        ],
        out_shape=[
            jax.ShapeDtypeStruct((N, NF), jnp.float32),
            jax.ShapeDtypeStruct((N, NF), jnp.float32),
            jax.ShapeDtypeStruct((N, 16), jnp.float32),
        ],
        compiler_params=pltpu.CompilerParams(
            dimension_semantics=("parallel",)),
    )(x, v2, w1a, w1b, b1r)


_GK = 64
_GCH = E // _GK
_GIT = -(-_GCH // NW)


def _gather_body(a2, b2, row, col, pxh, pyh, pzh, ar, bc, d2h,
                 idxr, idxc, bufa, bufb, px, py, pz, d2b, sema, semb):
    c = lax.axis_index("c")
    s = lax.axis_index("s")
    wid = s * NC + c
    pltpu.sync_copy(pxh, px)
    pltpu.sync_copy(pyh, py)
    pltpu.sync_copy(pzh, pz)

    def it(i, carry):
        cid = wid + NW * i

        @pl.when(cid < _GCH)
        def _():
            off = cid * _GK
            pltpu.sync_copy(row.at[pl.ds(off, _GK)], idxr)
            pltpu.sync_copy(col.at[pl.ds(off, _GK)], idxc)
            da = pltpu.async_copy(a2.at[idxr], bufa, sema)
            db = pltpu.async_copy(b2.at[idxc], bufb, semb)
            for j in range(_GK // 16):
                ir = idxr[pl.ds(j * 16, 16)]
                ic = idxc[pl.ds(j * 16, 16)]
                dx = plsc.load_gather(px, [ir]) - plsc.load_gather(px, [ic])
                dy = plsc.load_gather(py, [ir]) - plsc.load_gather(py, [ic])
                dz = plsc.load_gather(pz, [ir]) - plsc.load_gather(pz, [ic])
                d2b[pl.ds(j * 16, 16)] = dx * dx + dy * dy + dz * dz
            pltpu.sync_copy(d2b, d2h.at[pl.ds(off, _GK)])
            da.wait()
            pltpu.sync_copy(bufa, ar.at[pl.ds(off, _GK)])
            db.wait()
            pltpu.sync_copy(bufb, bc.at[pl.ds(off, _GK)])

        return carry

    lax.fori_loop(0, _GIT, it, 0)


def _sc_gather(a2, b2, row, col, px, py, pz):
    mesh = plsc.VectorSubcoreMesh(core_axis_name="c", subcore_axis_name="s")
    f = pl.kernel(
        _gather_body,
        out_type=[
            jax.ShapeDtypeStruct((E, NF), jnp.float32),
            jax.ShapeDtypeStruct((E, NF), jnp.float32),
            jax.ShapeDtypeStruct((E,), jnp.float32),
        ],
        mesh=mesh,
        scratch_types=[
            pltpu.VMEM((_GK,), jnp.int32),
            pltpu.VMEM((_GK,), jnp.int32),
            pltpu.VMEM((_GK, NF), jnp.float32),
            pltpu.VMEM((_GK, NF), jnp.float32),
            pltpu.VMEM((N,), jnp.float32),
            pltpu.VMEM((N,), jnp.float32),
            pltpu.VMEM((N,), jnp.float32),
            pltpu.VMEM((_GK,), jnp.float32),
            pltpu.SemaphoreType.DMA,
            pltpu.SemaphoreType.DMA,
        ],
        compiler_params=pltpu.CompilerParams(needs_layout_passes=False),
    )
    return f(a2, b2, row, col, px, py, pz)


_EB = 2000


def _mlp_body(ar_ref, bc_ref, d2_ref, w1d_ref, w2_ref, b2_ref, out_ref):
    dist = jnp.sqrt(d2_ref[...] + 1e-12)
    u = ar_ref[...] + bc_ref[...] + dist * w1d_ref[...]
    h = u * jax.nn.sigmoid(u)
    m = jnp.dot(h, w2_ref[...], preferred_element_type=jnp.float32) + b2_ref[...]
    msg = m * jax.nn.sigmoid(m)
    for k in range(4):
        out_ref[k, :, :] = msg[:, k * HD:(k + 1) * HD]


def _mlp(ar, bc, d2, w1d, w2, b2r):
    grid = (E // _EB,)
    return pl.pallas_call(
        _mlp_body,
        grid=grid,
        in_specs=[
            pl.BlockSpec((_EB, NF), lambda i: (i, 0)),
            pl.BlockSpec((_EB, NF), lambda i: (i, 0)),
            pl.BlockSpec((_EB, 1), lambda i: (i, 0)),
            pl.BlockSpec((1, NF), lambda i: (0, 0)),
            pl.BlockSpec((NF, NF), lambda i: (0, 0)),
            pl.BlockSpec((1, NF), lambda i: (0, 0)),
        ],
        out_specs=pl.BlockSpec((4, _EB, HD), lambda i: (0, i, 0)),
        out_shape=jax.ShapeDtypeStruct((4, E, HD), jnp.float32),
        compiler_params=pltpu.CompilerParams(
            dimension_semantics=("parallel",)),
    )(ar, bc, d2, w1d, w2, b2r)


_SK = 128
_SCH = E // _SK
_SIT = -(-_SCH // NS)
_RPT = 624
_TAIL = N - NS * _RPT


def _scatter_body(msg4, row, zr, agg4, idxv, mbuf, acc):
    c = lax.axis_index("c")
    s = lax.axis_index("s")
    for fc in range(2):
        fidx = c * 2 + fc
        pltpu.sync_copy(zr.at[pl.ds(0, _RPT)], acc.at[pl.ds(s * _RPT, _RPT)])

        @pl.when(s == 0)
        def _():
            pltpu.sync_copy(zr.at[pl.ds(0, _TAIL)],
                            acc.at[pl.ds(NS * _RPT, _TAIL)])

        plsc.subcore_barrier()

        def it(i, carry):
            cid = s + NS * i

            @pl.when(cid < _SCH)
            def _():
                off = cid * _SK
                pltpu.sync_copy(row.at[pl.ds(off, _SK)], idxv)
                pltpu.sync_copy(msg4.at[fidx, pl.ds(off, _SK), :], mbuf)
                pltpu.sync_copy(mbuf, acc.at[idxv], add=True)

            return carry

        lax.fori_loop(0, _SIT, it, 0)
        plsc.subcore_barrier()
        pltpu.sync_copy(acc.at[pl.ds(s * _RPT, _RPT)],
                        agg4.at[fidx, pl.ds(s * _RPT, _RPT), :])

        @pl.when(s == 0)
        def _():
            pltpu.sync_copy(acc.at[pl.ds(NS * _RPT, _TAIL)],
                            agg4.at[fidx, pl.ds(NS * _RPT, _TAIL), :])


def _sc_scatter(msg4, row):
    mesh = plsc.VectorSubcoreMesh(core_axis_name="c", subcore_axis_name="s")
    zr = jnp.zeros((_RPT, HD), jnp.float32)
    f = pl.kernel(
        _scatter_body,
        out_type=jax.ShapeDtypeStruct((4, N, HD), jnp.float32),
        mesh=mesh,
        scratch_types=[
            pltpu.VMEM((_SK,), jnp.int32),
            pltpu.VMEM((_SK, HD), jnp.float32),
            pltpu.VMEM_SHARED((N, HD), jnp.float32),
        ],
    )
    return f(msg4, row, zr)


def _final_body(x_ref, v_ref, agg_ref, u1a_ref, u1b_ref, c1_ref, u2_ref,
                c2_ref, out_ref):
    nf = jnp.concatenate([x_ref[...], v_ref[...]], axis=1)
    agg = jnp.concatenate([agg_ref[k] for k in range(4)], axis=1)
    g = (jnp.dot(nf, u1a_ref[...], preferred_element_type=jnp.float32)
         + jnp.dot(agg, u1b_ref[...], preferred_element_type=jnp.float32)
         + c1_ref[...])
    h2 = g * jax.nn.sigmoid(g)
    out_ref[...] = (jnp.dot(h2, u2_ref[...], preferred_element_type=jnp.float32)
                    + c2_ref[...])


def _final(x, v2, agg4, u1a, u1b, c1r, u2, c2r):
    grid = (N // _RB,)
    return pl.pallas_call(
        _final_body,
        grid=grid,
        in_specs=[
            pl.BlockSpec((_RB, HD), lambda i: (i, 0)),
            pl.BlockSpec((_RB, 3 * HD), lambda i: (i, 0)),
            pl.BlockSpec((4, _RB, HD), lambda i: (0, i, 0)),
            pl.BlockSpec((NF, NF), lambda i: (0, 0)),
            pl.BlockSpec((NF, NF), lambda i: (0, 0)),
            pl.BlockSpec((1, NF), lambda i: (0, 0)),
            pl.BlockSpec((NF, NF), lambda i: (0, 0)),
            pl.BlockSpec((1, NF), lambda i: (0, 0)),
        ],
        out_specs=pl.BlockSpec((_RB, NF), lambda i: (i, 0)),
        out_shape=jax.ShapeDtypeStruct((N, NF), jnp.float32),
        compiler_params=pltpu.CompilerParams(
            dimension_semantics=("parallel",)),
    )(x, v2, agg4, u1a, u1b, c1r, u2, c2r)


def kernel(x, v, edge_index, W1, b1, W2, b2, U1, c1, U2, c2):
    v2 = v.reshape(N, 3 * HD)
    row = edge_index[0].astype(jnp.int32)
    col = edge_index[1].astype(jnp.int32)
    w1a = W1[:NF]
    w1b = W1[NF:2 * NF]
    w1d = W1[2 * NF].reshape(1, NF)
    an, bn, pos16 = _pre(x, v2, w1a, w1b, b1.reshape(1, NF))
    px = pos16[:, 0]
    py = pos16[:, 1]
    pz = pos16[:, 2]
    ar, bc, d2 = _sc_gather(an, bn, row, col, px, py, pz)
    msg4 = _mlp(ar, bc, d2.reshape(E, 1), w1d, W2, b2.reshape(1, NF))
    agg4 = _sc_scatter(msg4, row)
    out = _final(x, v2, agg4, U1[:NF], U1[NF:], c1.reshape(1, NF),
                 U2, c2.reshape(1, NF))
    return out[:, :HD], out[:, HD:].reshape(N, 3, HD)
